# baseline (speedup 1.0000x reference)
import functools

import jax
import jax.numpy as jnp
from jax import lax
from jax.experimental import pallas as pl
from jax.experimental.pallas import tpu as pltpu
from jax.experimental.pallas import tpu_sc as plsc

N_NODES = 100000
HID = 128
NUM_CLASS = 32
NUM_SEG = 256

_R = 2048
_K = 23
_M = _R * _K
_C = 128
_NCH = (N_NODES - 32 - _M) // _C
_TAIL = N_NODES - _M - _NCH * _C
_NC = 2
_NS = 16
_NW = _NC * _NS
_MAXCH = (_NCH + _NW - 1) // _NW
_NLONG = _NCH - (_MAXCH - 1) * _NW
_IDPAD = _NW * _MAXCH


def _sc_body(x_hbm, ids2_hbm, ones_hbm, zc_hbm, sums_hbm, cnt_hbm,
             xbuf0, xbuf1, idbuf, ones_v, zer_v, zc_v, acc_sh, cnt_sh,
             sem_x, sem_i, sem_s, sem_c):
    cid = lax.axis_index("c")
    sid = lax.axis_index("s")
    wid = sid * _NC + cid

    extra = jnp.minimum(wid, _NLONG)
    c0 = wid * (_MAXCH - 1) + extra
    n = jnp.where(wid < _NLONG, _MAXCH, _MAXCH - 1)

    cp_ids = pltpu.async_copy(ids2_hbm.at[pl.ds(c0, _MAXCH)], idbuf, sem_i)
    pltpu.sync_copy(ones_hbm, ones_v)

    z16 = jnp.zeros((16,), jnp.float32)
    for r in range(16):
        for j in range(HID // 16):
            zer_v[r, pl.ds(j * 16, 16)] = z16
    pltpu.sync_copy(zer_v, acc_sh.at[pl.ds(sid * 16, 16)])

    @pl.when(sid == 0)
    def _zero_counts():
        pltpu.sync_copy(zc_hbm, zc_v)
        pltpu.sync_copy(zc_v, cnt_sh)

    cp_ids.wait()
    plsc.subcore_barrier()

    def x_chunk(i):
        return x_hbm.at[pl.ds(_M + (c0 + i) * _C, _C), :]

    def issue_load(i, buf):
        pltpu.async_copy(x_chunk(i), buf, sem_x)

    def wait_load(i, buf):
        pltpu.make_async_copy(x_chunk(i), buf, sem_x).wait()

    def scatter(i, buf):
        cp_s = pltpu.async_copy(buf, acc_sh.at[idbuf.at[i, 0]], sem_s, add=True)
        pltpu.async_copy(ones_v, cnt_sh.at[idbuf.at[i, 0]], sem_c, add=True).wait()
        cp_s.wait()

    issue_load(0, xbuf0)
    npairs = (_MAXCH + 1) // 2

    def pair_step(p, carry):
        i0 = 2 * p
        i1 = i0 + 1

        @pl.when(i0 < n)
        def _even():
            wait_load(i0, xbuf0)

            @pl.when(i1 < n)
            def _pf1():
                issue_load(i1, xbuf1)

            scatter(i0, xbuf0)

            @pl.when(i1 < n)
            def _odd():
                wait_load(i1, xbuf1)

                @pl.when(i1 + 1 < n)
                def _pf0():
                    issue_load(i1 + 1, xbuf0)

                scatter(i1, xbuf1)

        return carry

    lax.fori_loop(0, npairs, pair_step, 0)

    plsc.subcore_barrier()
    pltpu.sync_copy(acc_sh.at[pl.ds(sid * 16, 16)],
                    sums_hbm.at[cid, pl.ds(sid * 16, 16)])

    @pl.when(sid == 0)
    def _pub_counts():
        pltpu.sync_copy(cnt_sh, cnt_hbm.at[cid])


_sc_segsum = functools.partial(
    pl.kernel,
    out_type=(jax.ShapeDtypeStruct((_NC, NUM_SEG, HID), jnp.float32),
              jax.ShapeDtypeStruct((_NC, NUM_SEG), jnp.float32)),
    mesh=plsc.VectorSubcoreMesh(core_axis_name="c", subcore_axis_name="s",
                                num_cores=_NC, num_subcores=_NS),
    scratch_types=[
        pltpu.VMEM((_C, HID), jnp.float32),
        pltpu.VMEM((_C, HID), jnp.float32),
        pltpu.VMEM((_MAXCH, 1, _C), jnp.int32),
        pltpu.VMEM((_C,), jnp.float32),
        pltpu.VMEM((16, HID), jnp.float32),
        pltpu.VMEM((NUM_SEG,), jnp.float32),
        pltpu.VMEM_SHARED((NUM_SEG, HID), jnp.float32),
        pltpu.VMEM_SHARED((NUM_SEG,), jnp.float32),
        pltpu.SemaphoreType.DMA,
        pltpu.SemaphoreType.DMA,
        pltpu.SemaphoreType.DMA,
        pltpu.SemaphoreType.DMA,
    ],
)(_sc_body)


def _tc1_body(ids_ref, x_ref, sums_ref, cnt_ref):
    k = pl.program_id(0)

    @pl.when(k == 0)
    def _init():
        sums_ref[...] = jnp.zeros_like(sums_ref)
        cnt_ref[...] = jnp.zeros_like(cnt_ref)

    ids = ids_ref[0, 0, :]
    seg_iota = jax.lax.broadcasted_iota(jnp.int32, (NUM_SEG, _R), 0)
    onehot = (ids[None, :] == seg_iota).astype(jnp.float32)
    sums_ref[...] += jax.lax.dot(onehot, x_ref[...],
                                 preferred_element_type=jnp.float32)
    cnt_ref[0, :] += jnp.sum(onehot, axis=1)


def _tc1(ids3, x_front):
    return pl.pallas_call(
        _tc1_body,
        grid=(_K,),
        in_specs=[
            pl.BlockSpec((1, 1, _R), lambda k: (k, 0, 0)),
            pl.BlockSpec((_R, HID), lambda k: (k, 0)),
        ],
        out_specs=[
            pl.BlockSpec((NUM_SEG, HID), lambda k: (0, 0)),
            pl.BlockSpec((1, NUM_SEG), lambda k: (0, 0)),
        ],
        out_shape=[
            jax.ShapeDtypeStruct((NUM_SEG, HID), jnp.float32),
            jax.ShapeDtypeStruct((1, NUM_SEG), jnp.float32),
        ],
    )(ids3, x_front)


def _tc2_body(sums_tc_ref, cnt_tc_ref, sums_sc_ref, cnt_sc_ref,
              xt_ref, idt_ref, emb_ref, W_ref, b_ref, out_ref):
    sums = sums_tc_ref[...] + sums_sc_ref[0] + sums_sc_ref[1]
    seg_iota = jax.lax.broadcasted_iota(jnp.int32, (NUM_SEG, _TAIL), 0)
    onehot_t = (idt_ref[...] == seg_iota.astype(jnp.float32)).astype(jnp.float32)
    sums = sums + jax.lax.dot(onehot_t, xt_ref[...],
                              preferred_element_type=jnp.float32)
    counts = (cnt_tc_ref[0, :] + cnt_sc_ref[0] + cnt_sc_ref[1]
              + jnp.sum(onehot_t, axis=1))
    mean = sums / jnp.maximum(counts, 1.0)[:, None]
    cat = jnp.concatenate([mean, emb_ref[...]], axis=1)
    out_ref[...] = jax.lax.dot_general(
        cat, W_ref[...], (((1,), (1,)), ((), ())),
        preferred_element_type=jnp.float32) + b_ref[...]


def kernel(x, segment_ids, emb, W, b):
    ids = segment_ids.astype(jnp.int32)
    ids2 = jnp.zeros((_IDPAD, 1, _C), jnp.int32)
    ids2 = lax.dynamic_update_slice(
        ids2, ids[_M:_M + _NCH * _C].reshape(_NCH, 1, _C), (0, 0, 0))
    ones_c = jnp.ones((_C,), jnp.float32)
    zc_c = jnp.zeros((NUM_SEG,), jnp.float32)
    sums_sc, cnt_sc = _sc_segsum(x, ids2, ones_c, zc_c)

    ids3 = ids[:_M].reshape(_K, 1, _R)
    x_front = lax.slice(x, (0, 0), (_M, HID))
    sums_tc, cnt_tc = _tc1(ids3, x_front)

    x_tail = lax.slice(x, (_M + _NCH * _C, 0), (N_NODES, HID))
    ids_tail = ids[_M + _NCH * _C:].astype(jnp.float32).reshape(_TAIL, 1)
    out = pl.pallas_call(
        _tc2_body,
        out_shape=jax.ShapeDtypeStruct((NUM_SEG, NUM_CLASS), jnp.float32),
    )(sums_tc, cnt_tc, sums_sc, cnt_sc, x_tail, jnp.transpose(ids_tail),
      emb, W, b.reshape(1, NUM_CLASS))
    return out


# speedup vs baseline: 1.4047x; 1.4047x over previous
import functools

import jax
import jax.numpy as jnp
from jax import lax
from jax.experimental import pallas as pl
from jax.experimental.pallas import tpu as pltpu
from jax.experimental.pallas import tpu_sc as plsc

N_NODES = 100000
HID = 128
NUM_CLASS = 32
NUM_SEG = 256

_R = 2048
_K = 23
_M = _R * _K
_C = 128
_NCH = (N_NODES - 32 - _M) // _C
_TAIL = N_NODES - _M - _NCH * _C
_NC = 2
_NS = 16
_NW = _NC * _NS
_MAXCH = (_NCH + _NW - 1) // _NW
_NLONG = _NCH - (_MAXCH - 1) * _NW
_IDPAD = _NW * _MAXCH


def _sc_body(x_hbm, ids2_hbm, ones_hbm, zc_hbm, sums_hbm, cnt_hbm,
             xbuf0, xbuf1, xbuf2, xbuf3, idbuf, ones_v, zer_v, zc_v,
             acc_sh, cnt_sh,
             sem_x, sem_i, sem_s, sem_c):
    cid = lax.axis_index("c")
    sid = lax.axis_index("s")
    wid = sid * _NC + cid

    extra = jnp.minimum(wid, _NLONG)
    c0 = wid * (_MAXCH - 1) + extra
    n = jnp.where(wid < _NLONG, _MAXCH, _MAXCH - 1)

    cp_ids = pltpu.async_copy(ids2_hbm.at[pl.ds(c0, _MAXCH)], idbuf, sem_i)
    pltpu.sync_copy(ones_hbm, ones_v)

    z16 = jnp.zeros((16,), jnp.float32)
    for r in range(16):
        for j in range(HID // 16):
            zer_v[r, pl.ds(j * 16, 16)] = z16
    pltpu.sync_copy(zer_v, acc_sh.at[pl.ds(sid * 16, 16)])

    @pl.when(sid == 0)
    def _zero_counts():
        pltpu.sync_copy(zc_hbm, zc_v)
        pltpu.sync_copy(zc_v, cnt_sh)

    cp_ids.wait()
    plsc.subcore_barrier()

    bufs = (xbuf0, xbuf1, xbuf2, xbuf3)

    def x_chunk(i):
        return x_hbm.at[pl.ds(_M + (c0 + i) * _C, _C), :]

    def issue_load(i, buf):
        pltpu.async_copy(x_chunk(i), buf, sem_x)

    def wait_load(i, buf):
        pltpu.make_async_copy(x_chunk(i), buf, sem_x).wait()

    def issue_scatter(i, buf):
        pltpu.async_copy(buf, acc_sh.at[idbuf.at[i, 0]], sem_s, add=True)
        pltpu.async_copy(ones_v, cnt_sh.at[idbuf.at[i, 0]], sem_c, add=True)

    def drain_scatter():
        pltpu.make_async_copy(xbuf0, acc_sh.at[idbuf.at[0, 0]], sem_s).wait()
        pltpu.make_async_copy(ones_v, cnt_sh.at[idbuf.at[0, 0]], sem_c).wait()

    issue_load(0, xbuf0)
    issue_load(1, xbuf1)
    nquads = (_MAXCH + 3) // 4

    def quad_step(p, carry):
        for q in range(4):
            i = 4 * p + q

            @pl.when(i < n)
            def _body():
                wait_load(i, bufs[q])
                issue_scatter(i, bufs[q])

                @pl.when(i >= 2)
                def _drain():
                    drain_scatter()

                @pl.when(i + 2 < n)
                def _pf():
                    issue_load(i + 2, bufs[(q + 2) % 4])

        return carry

    lax.fori_loop(0, nquads, quad_step, 0)
    drain_scatter()
    drain_scatter()

    plsc.subcore_barrier()
    pltpu.sync_copy(acc_sh.at[pl.ds(sid * 16, 16)],
                    sums_hbm.at[cid, pl.ds(sid * 16, 16)])

    @pl.when(sid == 0)
    def _pub_counts():
        pltpu.sync_copy(cnt_sh, cnt_hbm.at[cid])


_sc_segsum = functools.partial(
    pl.kernel,
    out_type=(jax.ShapeDtypeStruct((_NC, NUM_SEG, HID), jnp.float32),
              jax.ShapeDtypeStruct((_NC, NUM_SEG), jnp.float32)),
    mesh=plsc.VectorSubcoreMesh(core_axis_name="c", subcore_axis_name="s",
                                num_cores=_NC, num_subcores=_NS),
    scratch_types=[
        pltpu.VMEM((_C, HID), jnp.float32),
        pltpu.VMEM((_C, HID), jnp.float32),
        pltpu.VMEM((_C, HID), jnp.float32),
        pltpu.VMEM((_C, HID), jnp.float32),
        pltpu.VMEM((_MAXCH, 1, _C), jnp.int32),
        pltpu.VMEM((_C,), jnp.float32),
        pltpu.VMEM((16, HID), jnp.float32),
        pltpu.VMEM((NUM_SEG,), jnp.float32),
        pltpu.VMEM_SHARED((NUM_SEG, HID), jnp.float32),
        pltpu.VMEM_SHARED((NUM_SEG,), jnp.float32),
        pltpu.SemaphoreType.DMA,
        pltpu.SemaphoreType.DMA,
        pltpu.SemaphoreType.DMA,
        pltpu.SemaphoreType.DMA,
    ],
)(_sc_body)


def _tc1_body(ids_ref, x_ref, sums_ref, cnt_ref):
    k = pl.program_id(0)

    @pl.when(k == 0)
    def _init():
        sums_ref[...] = jnp.zeros_like(sums_ref)
        cnt_ref[...] = jnp.zeros_like(cnt_ref)

    ids = ids_ref[0, 0, :]
    seg_iota = jax.lax.broadcasted_iota(jnp.int32, (NUM_SEG, _R), 0)
    onehot = (ids[None, :] == seg_iota).astype(jnp.float32)
    sums_ref[...] += jax.lax.dot(onehot, x_ref[...],
                                 preferred_element_type=jnp.float32)
    cnt_ref[0, :] += jnp.sum(onehot, axis=1)


def _tc1(ids3, x_front):
    return pl.pallas_call(
        _tc1_body,
        grid=(_K,),
        in_specs=[
            pl.BlockSpec((1, 1, _R), lambda k: (k, 0, 0)),
            pl.BlockSpec((_R, HID), lambda k: (k, 0)),
        ],
        out_specs=[
            pl.BlockSpec((NUM_SEG, HID), lambda k: (0, 0)),
            pl.BlockSpec((1, NUM_SEG), lambda k: (0, 0)),
        ],
        out_shape=[
            jax.ShapeDtypeStruct((NUM_SEG, HID), jnp.float32),
            jax.ShapeDtypeStruct((1, NUM_SEG), jnp.float32),
        ],
    )(ids3, x_front)


def _tc2_body(sums_tc_ref, cnt_tc_ref, sums_sc_ref, cnt_sc_ref,
              xt_ref, idt_ref, emb_ref, W_ref, b_ref, out_ref):
    sums = sums_tc_ref[...] + sums_sc_ref[0] + sums_sc_ref[1]
    seg_iota = jax.lax.broadcasted_iota(jnp.int32, (NUM_SEG, _TAIL), 0)
    onehot_t = (idt_ref[...] == seg_iota.astype(jnp.float32)).astype(jnp.float32)
    sums = sums + jax.lax.dot(onehot_t, xt_ref[...],
                              preferred_element_type=jnp.float32)
    counts = (cnt_tc_ref[0, :] + cnt_sc_ref[0] + cnt_sc_ref[1]
              + jnp.sum(onehot_t, axis=1))
    mean = sums / jnp.maximum(counts, 1.0)[:, None]
    cat = jnp.concatenate([mean, emb_ref[...]], axis=1)
    out_ref[...] = jax.lax.dot_general(
        cat, W_ref[...], (((1,), (1,)), ((), ())),
        preferred_element_type=jnp.float32) + b_ref[...]


def kernel(x, segment_ids, emb, W, b):
    ids = segment_ids.astype(jnp.int32)
    ids2 = jnp.zeros((_IDPAD, 1, _C), jnp.int32)
    ids2 = lax.dynamic_update_slice(
        ids2, ids[_M:_M + _NCH * _C].reshape(_NCH, 1, _C), (0, 0, 0))
    ones_c = jnp.ones((_C,), jnp.float32)
    zc_c = jnp.zeros((NUM_SEG,), jnp.float32)
    sums_sc, cnt_sc = _sc_segsum(x, ids2, ones_c, zc_c)

    ids3 = ids[:_M].reshape(_K, 1, _R)
    sums_tc, cnt_tc = _tc1(ids3, x)

    ids_tail = ids[_M + _NCH * _C:].astype(jnp.float32).reshape(_TAIL, 1)
    nblk = N_NODES // _TAIL - 1
    out = pl.pallas_call(
        _tc2_body,
        grid=(1,),
        in_specs=[
            pl.BlockSpec((NUM_SEG, HID), lambda k: (0, 0)),
            pl.BlockSpec((1, NUM_SEG), lambda k: (0, 0)),
            pl.BlockSpec((_NC, NUM_SEG, HID), lambda k: (0, 0, 0)),
            pl.BlockSpec((_NC, NUM_SEG), lambda k: (0, 0)),
            pl.BlockSpec((_TAIL, HID), lambda k: (nblk, 0)),
            pl.BlockSpec((1, _TAIL), lambda k: (0, 0)),
            pl.BlockSpec((NUM_SEG, HID), lambda k: (0, 0)),
            pl.BlockSpec((NUM_CLASS, 2 * HID), lambda k: (0, 0)),
            pl.BlockSpec((1, NUM_CLASS), lambda k: (0, 0)),
        ],
        out_specs=pl.BlockSpec((NUM_SEG, NUM_CLASS), lambda k: (0, 0)),
        out_shape=jax.ShapeDtypeStruct((NUM_SEG, NUM_CLASS), jnp.float32),
    )(sums_tc, cnt_tc, sums_sc, cnt_sc, x, jnp.transpose(ids_tail),
      emb, W, b.reshape(1, NUM_CLASS))
    return out


# speedup vs baseline: 1.5548x; 1.1069x over previous
import functools

import jax
import jax.numpy as jnp
from jax import lax
from jax.experimental import pallas as pl
from jax.experimental.pallas import tpu as pltpu
from jax.experimental.pallas import tpu_sc as plsc

N_NODES = 100000
HID = 128
NUM_CLASS = 32
NUM_SEG = 256

_R = 2920
_K = 16
_M = _R * _K
_C = 128
_NCH = (N_NODES - 32 - _M) // _C
_TAIL = N_NODES - _M - _NCH * _C
_NC = 2
_NS = 16
_NW = _NC * _NS
_MAXCH = _NCH // _NW


def _sc_body(x_hbm, ids2_hbm, ones_hbm, zc_hbm, sums_hbm, cnt_hbm,
             xbuf0, xbuf1, xbuf2, xbuf3, idbuf, ones_v, zer_v, zc_v,
             acc_sh, cnt_sh,
             sem_x, sem_i, sem_s, sem_c):
    cid = lax.axis_index("c")
    sid = lax.axis_index("s")
    wid = sid * _NC + cid

    c0 = wid * _MAXCH

    cp_ids = pltpu.async_copy(ids2_hbm.at[pl.ds(c0, _MAXCH)], idbuf, sem_i)
    pltpu.sync_copy(ones_hbm, ones_v)

    z16 = jnp.zeros((16,), jnp.float32)
    for r in range(16):
        for j in range(HID // 16):
            zer_v[r, pl.ds(j * 16, 16)] = z16
    pltpu.sync_copy(zer_v, acc_sh.at[pl.ds(sid * 16, 16)])

    @pl.when(sid == 0)
    def _zero_counts():
        pltpu.sync_copy(zc_hbm, zc_v)
        pltpu.sync_copy(zc_v, cnt_sh)

    cp_ids.wait()
    plsc.subcore_barrier()

    bufs = (xbuf0, xbuf1, xbuf2, xbuf3)

    def x_chunk(i):
        return x_hbm.at[pl.ds(_M + (c0 + i) * _C, _C), :]

    def issue_load(i, buf):
        pltpu.async_copy(x_chunk(i), buf, sem_x)

    def wait_load(i, buf):
        pltpu.make_async_copy(x_chunk(i), buf, sem_x).wait()

    def issue_scatter(i, buf):
        pltpu.async_copy(buf, acc_sh.at[idbuf.at[i, 0]], sem_s, add=True)
        pltpu.async_copy(ones_v, cnt_sh.at[idbuf.at[i, 0]], sem_c, add=True)

    def drain_scatter():
        pltpu.make_async_copy(xbuf0, acc_sh.at[idbuf.at[0, 0]], sem_s).wait()
        pltpu.make_async_copy(ones_v, cnt_sh.at[idbuf.at[0, 0]], sem_c).wait()

    issue_load(0, xbuf0)
    issue_load(1, xbuf1)
    for i in range(_MAXCH):
        wait_load(i, bufs[i % 4])
        issue_scatter(i, bufs[i % 4])
        if i >= 2:
            drain_scatter()
        if i + 2 < _MAXCH:
            issue_load(i + 2, bufs[(i + 2) % 4])
    drain_scatter()
    drain_scatter()

    plsc.subcore_barrier()
    pltpu.sync_copy(acc_sh.at[pl.ds(sid * 16, 16)],
                    sums_hbm.at[cid, pl.ds(sid * 16, 16)])

    @pl.when(sid == 0)
    def _pub_counts():
        pltpu.sync_copy(cnt_sh, cnt_hbm.at[cid])


_sc_segsum = functools.partial(
    pl.kernel,
    out_type=(jax.ShapeDtypeStruct((_NC, NUM_SEG, HID), jnp.float32),
              jax.ShapeDtypeStruct((_NC, NUM_SEG), jnp.float32)),
    mesh=plsc.VectorSubcoreMesh(core_axis_name="c", subcore_axis_name="s",
                                num_cores=_NC, num_subcores=_NS),
    scratch_types=[
        pltpu.VMEM((_C, HID), jnp.float32),
        pltpu.VMEM((_C, HID), jnp.float32),
        pltpu.VMEM((_C, HID), jnp.float32),
        pltpu.VMEM((_C, HID), jnp.float32),
        pltpu.VMEM((_MAXCH, 1, _C), jnp.int32),
        pltpu.VMEM((_C,), jnp.float32),
        pltpu.VMEM((16, HID), jnp.float32),
        pltpu.VMEM((NUM_SEG,), jnp.float32),
        pltpu.VMEM_SHARED((NUM_SEG, HID), jnp.float32),
        pltpu.VMEM_SHARED((NUM_SEG,), jnp.float32),
        pltpu.SemaphoreType.DMA,
        pltpu.SemaphoreType.DMA,
        pltpu.SemaphoreType.DMA,
        pltpu.SemaphoreType.DMA,
    ],
)(_sc_body)


def _tc1_body(ids_ref, x_ref, sums_ref, cnt_ref):
    k = pl.program_id(0)

    @pl.when(k == 0)
    def _init():
        sums_ref[...] = jnp.zeros_like(sums_ref)
        cnt_ref[...] = jnp.zeros_like(cnt_ref)

    ids = ids_ref[0, 0, :]
    seg_iota = jax.lax.broadcasted_iota(jnp.int32, (NUM_SEG, _R), 0)
    onehot = (ids[None, :] == seg_iota).astype(jnp.bfloat16)
    sums_ref[...] += jax.lax.dot(onehot, x_ref[...].astype(jnp.bfloat16),
                                 preferred_element_type=jnp.float32)
    cnt_ref[0, :] += jnp.sum(onehot.astype(jnp.float32), axis=1)


def _tc1(ids3, x_front):
    return pl.pallas_call(
        _tc1_body,
        grid=(_K,),
        in_specs=[
            pl.BlockSpec((1, 1, _R), lambda k: (k, 0, 0)),
            pl.BlockSpec((_R, HID), lambda k: (k, 0)),
        ],
        out_specs=[
            pl.BlockSpec((NUM_SEG, HID), lambda k: (0, 0)),
            pl.BlockSpec((1, NUM_SEG), lambda k: (0, 0)),
        ],
        out_shape=[
            jax.ShapeDtypeStruct((NUM_SEG, HID), jnp.float32),
            jax.ShapeDtypeStruct((1, NUM_SEG), jnp.float32),
        ],
    )(ids3, x_front)


def _tc2_body(sums_tc_ref, cnt_tc_ref, sums_sc_ref, cnt_sc_ref,
              xt_ref, idt_ref, emb_ref, W_ref, b_ref, out_ref):
    sums = sums_tc_ref[...] + sums_sc_ref[0] + sums_sc_ref[1]
    seg_iota = jax.lax.broadcasted_iota(jnp.int32, (NUM_SEG, _TAIL), 0)
    onehot_t = (idt_ref[...] == seg_iota.astype(jnp.float32)).astype(jnp.float32)
    sums = sums + jax.lax.dot(onehot_t, xt_ref[...],
                              preferred_element_type=jnp.float32)
    counts = (cnt_tc_ref[0, :] + cnt_sc_ref[0] + cnt_sc_ref[1]
              + jnp.sum(onehot_t, axis=1))
    mean = sums / jnp.maximum(counts, 1.0)[:, None]
    cat = jnp.concatenate([mean, emb_ref[...]], axis=1)
    out_ref[...] = jax.lax.dot_general(
        cat, W_ref[...], (((1,), (1,)), ((), ())),
        preferred_element_type=jnp.float32) + b_ref[...]


def kernel(x, segment_ids, emb, W, b):
    ids = segment_ids.astype(jnp.int32)
    ids2 = ids[_M:_M + _NCH * _C].reshape(_NCH, 1, _C)
    ones_c = jnp.ones((_C,), jnp.float32)
    zc_c = jnp.zeros((NUM_SEG,), jnp.float32)
    sums_sc, cnt_sc = _sc_segsum(x, ids2, ones_c, zc_c)

    ids3 = ids[:_M].reshape(_K, 1, _R)
    sums_tc, cnt_tc = _tc1(ids3, x)

    ids_tail = ids[_M + _NCH * _C:].astype(jnp.float32).reshape(_TAIL, 1)
    nblk = N_NODES // _TAIL - 1
    out = pl.pallas_call(
        _tc2_body,
        grid=(1,),
        in_specs=[
            pl.BlockSpec((NUM_SEG, HID), lambda k: (0, 0)),
            pl.BlockSpec((1, NUM_SEG), lambda k: (0, 0)),
            pl.BlockSpec((_NC, NUM_SEG, HID), lambda k: (0, 0, 0)),
            pl.BlockSpec((_NC, NUM_SEG), lambda k: (0, 0)),
            pl.BlockSpec((_TAIL, HID), lambda k: (nblk, 0)),
            pl.BlockSpec((1, _TAIL), lambda k: (0, 0)),
            pl.BlockSpec((NUM_SEG, HID), lambda k: (0, 0)),
            pl.BlockSpec((NUM_CLASS, 2 * HID), lambda k: (0, 0)),
            pl.BlockSpec((1, NUM_CLASS), lambda k: (0, 0)),
        ],
        out_specs=pl.BlockSpec((NUM_SEG, NUM_CLASS), lambda k: (0, 0)),
        out_shape=jax.ShapeDtypeStruct((NUM_SEG, NUM_CLASS), jnp.float32),
    )(sums_tc, cnt_tc, sums_sc, cnt_sc, x, jnp.transpose(ids_tail),
      emb, W, b.reshape(1, NUM_CLASS))
    return out
